# baseline (speedup 1.0000x reference)
import functools

import jax
import jax.numpy as jnp
from jax import lax
from jax.experimental import pallas as pl
from jax.experimental.pallas import tpu as pltpu
from jax.experimental.pallas import tpu_sc as plsc

D = 128
LANG_PAD = 104


def _project_body(char_ref, lang_ref, w_ref, b_ref, cout_ref, lout_ref):
    w = w_ref[...]
    w1 = w[:, :D]
    w2 = w[:, D:]
    cn = (((1,), (1,)), ((), ()))
    cout_ref[...] = (
        lax.dot_general(char_ref[...], w1, cn, preferred_element_type=jnp.float32)
        + b_ref[...]
    )
    lout_ref[...] = lax.dot_general(
        lang_ref[...], w2, cn, preferred_element_type=jnp.float32
    )


def _project(char_table, lang_table_padded, W, b2d):
    n_chars = char_table.shape[0]
    return pl.pallas_call(
        _project_body,
        out_shape=[
            jax.ShapeDtypeStruct((n_chars, D), jnp.float32),
            jax.ShapeDtypeStruct((LANG_PAD, D), jnp.float32),
        ],
    )(char_table, lang_table_padded, W, b2d)


def _make_sc_gather(n_total):
    info = plsc.get_sparse_core_info()
    nw = info.num_cores * info.num_subcores
    per_w = n_total // nw
    ch = 128
    n_ch = per_w // ch
    nbuf = 4
    mesh = plsc.VectorSubcoreMesh(core_axis_name="c", subcore_axis_name="s")

    @functools.partial(
        pl.kernel,
        mesh=mesh,
        out_type=jax.ShapeDtypeStruct((n_total, D), jnp.float32),
        scratch_types=[
            pltpu.VMEM((per_w,), jnp.int32),
            pltpu.VMEM((ch, D), jnp.float32),
            pltpu.VMEM((ch, D), jnp.float32),
            pltpu.VMEM((ch, D), jnp.float32),
            pltpu.VMEM((ch, D), jnp.float32),
            pltpu.SemaphoreType.DMA,
            pltpu.SemaphoreType.DMA,
            pltpu.SemaphoreType.DMA,
            pltpu.SemaphoreType.DMA,
            pltpu.SemaphoreType.DMA,
            pltpu.SemaphoreType.DMA,
            pltpu.SemaphoreType.DMA,
            pltpu.SemaphoreType.DMA,
        ],
    )
    def sc_gather(cproj_hbm, ci_hbm, out_hbm,
                  ci_v, ga0, ga1, ga2, ga3,
                  gs0, gs1, gs2, gs3, ss0, ss1, ss2, ss3):
        wid = lax.axis_index("s") * info.num_cores + lax.axis_index("c")
        base = wid * per_w
        pltpu.sync_copy(ci_hbm.at[pl.ds(base, per_w)], ci_v)
        gbufs = (ga0, ga1, ga2, ga3)
        gsems = (gs0, gs1, gs2, gs3)
        ssems = (ss0, ss1, ss2, ss3)

        def start_gather(g, b):
            pltpu.async_copy(
                cproj_hbm.at[ci_v.at[pl.ds(g * ch, ch)]], gbufs[b], gsems[b]
            )

        def wait_gather(b):
            pltpu.make_async_copy(
                cproj_hbm.at[ci_v.at[pl.ds(0, ch)]], gbufs[b], gsems[b]
            ).wait()

        def start_scatter(g, b):
            pltpu.async_copy(
                gbufs[b], out_hbm.at[pl.ds(base + g * ch, ch)], ssems[b]
            )

        def wait_scatter(b):
            pltpu.make_async_copy(
                gbufs[b], out_hbm.at[pl.ds(base, ch)], ssems[b]
            ).wait()

        for g in range(3):
            start_gather(g, g % nbuf)
        for t in range(n_ch):
            b = t % nbuf
            if t + 3 < n_ch:
                bn = (t + 3) % nbuf
                if t - 1 >= 0:
                    wait_scatter(bn)
                start_gather(t + 3, bn)
            wait_gather(b)
            start_scatter(t, b)
        for b in range(nbuf):
            wait_scatter(b)

    return sc_gather


_BB = 32


def _finish_body(part_ref, li_ref, lproj_ref, out_ref):
    L = li_ref.shape[1]
    li = li_ref[...]
    oh = (
        li[..., None] == lax.broadcasted_iota(jnp.int32, (_BB, L, LANG_PAD), 2)
    ).astype(jnp.float32)
    lang_rows = lax.dot_general(
        oh, lproj_ref[...], (((2,), (0,)), ((), ())),
        precision=lax.Precision.HIGHEST,
        preferred_element_type=jnp.float32,
    )
    out_ref[...] = part_ref[...].reshape(_BB, L, D) + lang_rows


def _finish(flat, lang_indices, lproj):
    B, L = lang_indices.shape
    return pl.pallas_call(
        _finish_body,
        grid=(B // _BB,),
        in_specs=[
            pl.BlockSpec((_BB * L, D), lambda i: (i, 0)),
            pl.BlockSpec((_BB, L), lambda i: (i, 0)),
            pl.BlockSpec((LANG_PAD, D), lambda i: (0, 0)),
        ],
        out_specs=pl.BlockSpec((_BB, L, D), lambda i: (i, 0, 0)),
        out_shape=jax.ShapeDtypeStruct((B, L, D), jnp.float32),
    )(flat, lang_indices, lproj)


def kernel(char_indices, lang_indices, char_table, lang_table, W, b):
    B, L = char_indices.shape
    n_total = B * L
    lang_padded = jnp.pad(lang_table, ((0, LANG_PAD - lang_table.shape[0]), (0, 0)))
    cproj, lproj = _project(char_table, lang_padded, W, b.reshape(1, D))
    ci = char_indices.reshape(-1).astype(jnp.int32)
    out = _make_sc_gather(n_total)(cproj, ci)
    return _finish(out, lang_indices.astype(jnp.int32), lproj)


# speedup vs baseline: 1.0829x; 1.0829x over previous
import functools

import jax
import jax.numpy as jnp
from jax import lax
from jax.experimental import pallas as pl
from jax.experimental.pallas import tpu as pltpu
from jax.experimental.pallas import tpu_sc as plsc

D = 128
LANG_PAD = 104


def _project_body(char_ref, lang_ref, w_ref, b_ref, cout_ref, lout_ref):
    w = w_ref[...]
    w1 = w[:, :D]
    w2 = w[:, D:]
    cn = (((1,), (1,)), ((), ()))
    cout_ref[...] = (
        lax.dot_general(char_ref[...], w1, cn, preferred_element_type=jnp.float32)
        + b_ref[...]
    )
    lout_ref[...] = lax.dot_general(
        lang_ref[...], w2, cn, preferred_element_type=jnp.float32
    )


def _project(char_table, lang_table_padded, W, b2d):
    n_chars = char_table.shape[0]
    return pl.pallas_call(
        _project_body,
        out_shape=[
            jax.ShapeDtypeStruct((n_chars, D), jnp.float32),
            jax.ShapeDtypeStruct((LANG_PAD, D), jnp.float32),
        ],
    )(char_table, lang_table_padded, W, b2d)


def _make_sc_gather(n_total):
    info = plsc.get_sparse_core_info()
    nw = info.num_cores * info.num_subcores
    per_w = n_total // nw
    ch = 128
    n_ch = per_w // ch
    nbuf = 4
    mesh = plsc.VectorSubcoreMesh(core_axis_name="c", subcore_axis_name="s")

    @functools.partial(
        pl.kernel,
        mesh=mesh,
        out_type=jax.ShapeDtypeStruct((n_total, D), jnp.float32),
        scratch_types=[
            pltpu.VMEM((per_w,), jnp.int32),
            pltpu.VMEM((ch, D), jnp.float32),
            pltpu.VMEM((ch, D), jnp.float32),
            pltpu.VMEM((ch, D), jnp.float32),
            pltpu.VMEM((ch, D), jnp.float32),
            pltpu.SemaphoreType.DMA,
            pltpu.SemaphoreType.DMA,
            pltpu.SemaphoreType.DMA,
            pltpu.SemaphoreType.DMA,
            pltpu.SemaphoreType.DMA,
            pltpu.SemaphoreType.DMA,
            pltpu.SemaphoreType.DMA,
            pltpu.SemaphoreType.DMA,
        ],
    )
    def sc_gather(cproj_hbm, ci_hbm, out_hbm,
                  ci_v, ga0, ga1, ga2, ga3,
                  gs0, gs1, gs2, gs3, ss0, ss1, ss2, ss3):
        wid = lax.axis_index("s") * info.num_cores + lax.axis_index("c")
        base = wid * per_w
        pltpu.sync_copy(ci_hbm.at[pl.ds(base, per_w)], ci_v)
        gbufs = (ga0, ga1, ga2, ga3)
        gsems = (gs0, gs1, gs2, gs3)
        ssems = (ss0, ss1, ss2, ss3)

        def start_gather(g, b):
            pltpu.async_copy(
                cproj_hbm.at[ci_v.at[pl.ds(g * ch, ch)]], gbufs[b], gsems[b]
            )

        def wait_gather(b):
            pltpu.make_async_copy(
                cproj_hbm.at[ci_v.at[pl.ds(0, ch)]], gbufs[b], gsems[b]
            ).wait()

        def start_scatter(g, b):
            pltpu.async_copy(
                gbufs[b], out_hbm.at[pl.ds(base + g * ch, ch)], ssems[b]
            )

        def wait_scatter(b):
            pltpu.make_async_copy(
                gbufs[b], out_hbm.at[pl.ds(base, ch)], ssems[b]
            ).wait()

        for g in range(3):
            start_gather(g, g % nbuf)
        for t in range(n_ch):
            b = t % nbuf
            if t + 3 < n_ch:
                bn = (t + 3) % nbuf
                if t - 1 >= 0:
                    wait_scatter(bn)
                start_gather(t + 3, bn)
            wait_gather(b)
            start_scatter(t, b)
        for b in range(nbuf):
            wait_scatter(b)

    return sc_gather


_BB = 32


def _finish_body(part_ref, li_ref, lproj_ref, out_ref):
    L = li_ref.shape[1]
    li = li_ref[...]
    oh = (
        li[..., None] == lax.broadcasted_iota(jnp.int32, (_BB, L, LANG_PAD), 2)
    ).astype(jnp.float32)
    lproj = lproj_ref[...]
    lproj_hi = lproj.astype(jnp.bfloat16).astype(jnp.float32)
    lproj_lo = lproj - lproj_hi
    cn = (((2,), (0,)), ((), ()))
    lang_rows = lax.dot_general(
        oh, lproj_hi, cn, preferred_element_type=jnp.float32
    ) + lax.dot_general(oh, lproj_lo, cn, preferred_element_type=jnp.float32)
    out_ref[...] = part_ref[...].reshape(_BB, L, D) + lang_rows


def _finish(flat, lang_indices, lproj):
    B, L = lang_indices.shape
    return pl.pallas_call(
        _finish_body,
        grid=(B // _BB,),
        in_specs=[
            pl.BlockSpec((_BB * L, D), lambda i: (i, 0)),
            pl.BlockSpec((_BB, L), lambda i: (i, 0)),
            pl.BlockSpec((LANG_PAD, D), lambda i: (0, 0)),
        ],
        out_specs=pl.BlockSpec((_BB, L, D), lambda i: (i, 0, 0)),
        out_shape=jax.ShapeDtypeStruct((B, L, D), jnp.float32),
    )(flat, lang_indices, lproj)


def kernel(char_indices, lang_indices, char_table, lang_table, W, b):
    B, L = char_indices.shape
    n_total = B * L
    lang_padded = jnp.pad(lang_table, ((0, LANG_PAD - lang_table.shape[0]), (0, 0)))
    cproj, lproj = _project(char_table, lang_padded, W, b.reshape(1, D))
    ci = char_indices.reshape(-1).astype(jnp.int32)
    out = _make_sc_gather(n_total)(cproj, ci)
    return _finish(out, lang_indices.astype(jnp.int32), lproj)


# speedup vs baseline: 1.1772x; 1.0871x over previous
import functools

import jax
import jax.numpy as jnp
from jax import lax
from jax.experimental import pallas as pl
from jax.experimental.pallas import tpu as pltpu
from jax.experimental.pallas import tpu_sc as plsc

D = 128
LANG_PAD = 104


def _project_body(char_ref, lang_ref, w_ref, b_ref, cout_ref, lout_ref):
    w = w_ref[...]
    w1 = w[:, :D]
    w2 = w[:, D:]
    cn = (((1,), (1,)), ((), ()))
    cout_ref[...] = (
        lax.dot_general(char_ref[...], w1, cn, preferred_element_type=jnp.float32)
        + b_ref[...]
    )
    lout_ref[...] = lax.dot_general(
        lang_ref[...], w2, cn, preferred_element_type=jnp.float32
    )


def _project(char_table, lang_table_padded, W, b2d):
    n_chars = char_table.shape[0]
    return pl.pallas_call(
        _project_body,
        out_shape=[
            jax.ShapeDtypeStruct((n_chars, D), jnp.float32),
            jax.ShapeDtypeStruct((LANG_PAD, D), jnp.float32),
        ],
    )(char_table, lang_table_padded, W, b2d)


def _make_sc_gather(n_total):
    info = plsc.get_sparse_core_info()
    nw = info.num_cores * info.num_subcores
    per_w = n_total // nw
    ch = 128
    n_ch = per_w // ch
    nbuf = 4
    mesh = plsc.VectorSubcoreMesh(core_axis_name="c", subcore_axis_name="s")

    @functools.partial(
        pl.kernel,
        mesh=mesh,
        out_type=jax.ShapeDtypeStruct((n_total, D), jnp.float32),
        scratch_types=[
            pltpu.VMEM((per_w,), jnp.int32),
            pltpu.VMEM((ch, D), jnp.float32),
            pltpu.VMEM((ch, D), jnp.float32),
            pltpu.VMEM((ch, D), jnp.float32),
            pltpu.VMEM((ch, D), jnp.float32),
            pltpu.SemaphoreType.DMA,
            pltpu.SemaphoreType.DMA,
            pltpu.SemaphoreType.DMA,
            pltpu.SemaphoreType.DMA,
            pltpu.SemaphoreType.DMA,
            pltpu.SemaphoreType.DMA,
            pltpu.SemaphoreType.DMA,
            pltpu.SemaphoreType.DMA,
        ],
    )
    def sc_gather(cproj_hbm, ci_hbm, out_hbm,
                  ci_v, ga0, ga1, ga2, ga3,
                  gs0, gs1, gs2, gs3, ss0, ss1, ss2, ss3):
        wid = lax.axis_index("s") * info.num_cores + lax.axis_index("c")
        base = wid * per_w
        pltpu.sync_copy(ci_hbm.at[pl.ds(base, per_w)], ci_v)
        gbufs = (ga0, ga1, ga2, ga3)
        gsems = (gs0, gs1, gs2, gs3)
        ssems = (ss0, ss1, ss2, ss3)

        def start_gather(g, b):
            pltpu.async_copy(
                cproj_hbm.at[ci_v.at[pl.ds(g * ch, ch)]], gbufs[b], gsems[b]
            )

        def wait_gather(b):
            pltpu.make_async_copy(
                cproj_hbm.at[ci_v.at[pl.ds(0, ch)]], gbufs[b], gsems[b]
            ).wait()

        def start_scatter(g, b):
            pltpu.async_copy(
                gbufs[b], out_hbm.at[pl.ds(base + g * ch, ch)], ssems[b]
            )

        def wait_scatter(b):
            pltpu.make_async_copy(
                gbufs[b], out_hbm.at[pl.ds(base, ch)], ssems[b]
            ).wait()

        for g in range(3):
            start_gather(g, g % nbuf)
        for t in range(n_ch):
            b = t % nbuf
            if t + 3 < n_ch:
                bn = (t + 3) % nbuf
                if t - 1 >= 0:
                    wait_scatter(bn)
                start_gather(t + 3, bn)
            wait_gather(b)
            start_scatter(t, b)
        for b in range(nbuf):
            wait_scatter(b)

    return sc_gather


_BB = 32


def _finish_math(part_ref, li_ref, lproj_ref, out_ref):
    L = li_ref.shape[1]
    li = li_ref[...]
    oh = (
        li[..., None] == lax.broadcasted_iota(jnp.int32, (_BB, L, LANG_PAD), 2)
    ).astype(jnp.float32)
    lproj = lproj_ref[...]
    lproj_hi = lproj.astype(jnp.bfloat16).astype(jnp.float32)
    lproj_lo = lproj - lproj_hi
    cn = (((2,), (0,)), ((), ()))
    lang_rows = lax.dot_general(
        oh, lproj_hi, cn, preferred_element_type=jnp.float32
    ) + lax.dot_general(oh, lproj_lo, cn, preferred_element_type=jnp.float32)
    out_ref[...] = part_ref[...].reshape(_BB, L, D) + lang_rows


def _finish_body0(part_ref, li_ref, lproj_ref, out_ref):
    _finish_math(part_ref, li_ref, lproj_ref, out_ref)


def _finish_body1(part_ref, li_ref, lproj_ref, prev_ref, out_ref):
    del prev_ref
    _finish_math(part_ref, li_ref, lproj_ref, out_ref)


def _finish_half(flat_half, li_half, lproj, B, L, half, prev=None):
    nblk = li_half.shape[0] // _BB
    off = half * nblk
    common = dict(
        grid=(nblk,),
        out_specs=pl.BlockSpec((_BB, L, D), lambda i: (i + off, 0, 0)),
        out_shape=jax.ShapeDtypeStruct((B, L, D), jnp.float32),
    )
    in_specs = [
        pl.BlockSpec((_BB * L, D), lambda i: (i, 0)),
        pl.BlockSpec((_BB, L), lambda i: (i, 0)),
        pl.BlockSpec((LANG_PAD, D), lambda i: (0, 0)),
    ]
    if prev is None:
        return pl.pallas_call(_finish_body0, in_specs=in_specs, **common)(
            flat_half, li_half, lproj
        )
    in_specs.append(pl.BlockSpec(memory_space=pl.ANY))
    return pl.pallas_call(
        _finish_body1,
        in_specs=in_specs,
        input_output_aliases={3: 0},
        **common,
    )(flat_half, li_half, lproj, prev)


def kernel(char_indices, lang_indices, char_table, lang_table, W, b):
    B, L = char_indices.shape
    n_total = B * L
    lang_padded = jnp.pad(lang_table, ((0, LANG_PAD - lang_table.shape[0]), (0, 0)))
    cproj, lproj = _project(char_table, lang_padded, W, b.reshape(1, D))
    ci = char_indices.reshape(-1).astype(jnp.int32)
    li = lang_indices.astype(jnp.int32)
    half_n = n_total // 2
    half_b = B // 2
    sc = _make_sc_gather(half_n)
    part0 = sc(cproj, ci[:half_n])
    part1 = sc(cproj, ci[half_n:])
    out0 = _finish_half(part0, li[:half_b], lproj, B, L, 0)
    out1 = _finish_half(part1, li[half_b:], lproj, B, L, 1, prev=out0)
    return out1


# speedup vs baseline: 1.2944x; 1.0995x over previous
import functools

import jax
import jax.numpy as jnp
from jax import lax
from jax.experimental import pallas as pl
from jax.experimental.pallas import tpu as pltpu
from jax.experimental.pallas import tpu_sc as plsc

D = 128
LANG_PAD = 104


def _project_body(char_ref, lang_ref, w_ref, b_ref, cout_ref, lout_ref):
    w = w_ref[...]
    w1 = w[:, :D]
    w2 = w[:, D:]
    cn = (((1,), (1,)), ((), ()))
    cout_ref[...] = (
        lax.dot_general(char_ref[...], w1, cn, preferred_element_type=jnp.float32)
        + b_ref[...]
    )
    lout_ref[...] = lax.dot_general(
        lang_ref[...], w2, cn, preferred_element_type=jnp.float32
    )


def _project(char_table, lang_table_padded, W, b2d):
    n_chars = char_table.shape[0]
    return pl.pallas_call(
        _project_body,
        out_shape=[
            jax.ShapeDtypeStruct((n_chars, D), jnp.float32),
            jax.ShapeDtypeStruct((LANG_PAD, D), jnp.float32),
        ],
    )(char_table, lang_table_padded, W, b2d)


def _make_sc_gather(n_total):
    info = plsc.get_sparse_core_info()
    nw = info.num_cores * info.num_subcores
    per_w = n_total // nw
    ch = 128
    n_ch = per_w // ch
    nbuf = 4
    mesh = plsc.VectorSubcoreMesh(core_axis_name="c", subcore_axis_name="s")

    @functools.partial(
        pl.kernel,
        mesh=mesh,
        out_type=jax.ShapeDtypeStruct((n_total, D), jnp.float32),
        scratch_types=[
            pltpu.VMEM((per_w,), jnp.int32),
            pltpu.VMEM((ch, D), jnp.float32),
            pltpu.VMEM((ch, D), jnp.float32),
            pltpu.VMEM((ch, D), jnp.float32),
            pltpu.VMEM((ch, D), jnp.float32),
            pltpu.SemaphoreType.DMA,
            pltpu.SemaphoreType.DMA,
            pltpu.SemaphoreType.DMA,
            pltpu.SemaphoreType.DMA,
            pltpu.SemaphoreType.DMA,
            pltpu.SemaphoreType.DMA,
            pltpu.SemaphoreType.DMA,
            pltpu.SemaphoreType.DMA,
        ],
    )
    def sc_gather(cproj_hbm, ci_hbm, out_hbm,
                  ci_v, ga0, ga1, ga2, ga3,
                  gs0, gs1, gs2, gs3, ss0, ss1, ss2, ss3):
        wid = lax.axis_index("s") * info.num_cores + lax.axis_index("c")
        base = wid * per_w
        pltpu.sync_copy(ci_hbm.at[pl.ds(base, per_w)], ci_v)
        gbufs = (ga0, ga1, ga2, ga3)
        gsems = (gs0, gs1, gs2, gs3)
        ssems = (ss0, ss1, ss2, ss3)

        def start_gather(g, b):
            pltpu.async_copy(
                cproj_hbm.at[ci_v.at[pl.ds(g * ch, ch)]], gbufs[b], gsems[b]
            )

        def wait_gather(b):
            pltpu.make_async_copy(
                cproj_hbm.at[ci_v.at[pl.ds(0, ch)]], gbufs[b], gsems[b]
            ).wait()

        def start_scatter(g, b):
            pltpu.async_copy(
                gbufs[b], out_hbm.at[pl.ds(base + g * ch, ch)], ssems[b]
            )

        def wait_scatter(b):
            pltpu.make_async_copy(
                gbufs[b], out_hbm.at[pl.ds(base, ch)], ssems[b]
            ).wait()

        for g in range(3):
            start_gather(g, g % nbuf)
        for t in range(n_ch):
            b = t % nbuf
            if t + 3 < n_ch:
                bn = (t + 3) % nbuf
                if t - 1 >= 0:
                    wait_scatter(bn)
                start_gather(t + 3, bn)
            wait_gather(b)
            start_scatter(t, b)
        for b in range(nbuf):
            wait_scatter(b)

    return sc_gather


_BB = 64


def _finish_math(part_ref, li_ref, lproj_ref, out_ref):
    L = li_ref.shape[1]
    li = li_ref[...]
    oh = (
        li[..., None] == lax.broadcasted_iota(jnp.int32, (_BB, L, LANG_PAD), 2)
    ).astype(jnp.float32)
    lproj = lproj_ref[...]
    lproj_hi = lproj.astype(jnp.bfloat16).astype(jnp.float32)
    lproj_lo = lproj - lproj_hi
    cn = (((2,), (0,)), ((), ()))
    lang_rows = lax.dot_general(
        oh, lproj_hi, cn, preferred_element_type=jnp.float32
    ) + lax.dot_general(oh, lproj_lo, cn, preferred_element_type=jnp.float32)
    out_ref[...] = part_ref[...].reshape(_BB, L, D) + lang_rows


def _finish_body0(part_ref, li_ref, lproj_ref, out_ref):
    _finish_math(part_ref, li_ref, lproj_ref, out_ref)


def _finish_body1(part_ref, li_ref, lproj_ref, prev_ref, out_ref):
    del prev_ref
    _finish_math(part_ref, li_ref, lproj_ref, out_ref)


def _finish_half(flat_half, li_half, lproj, B, L, half, prev=None):
    nblk = li_half.shape[0] // _BB
    off = half * nblk
    common = dict(
        grid=(nblk,),
        out_specs=pl.BlockSpec((_BB, L, D), lambda i: (i + off, 0, 0)),
        out_shape=jax.ShapeDtypeStruct((B, L, D), jnp.float32),
    )
    in_specs = [
        pl.BlockSpec((_BB * L, D), lambda i: (i, 0)),
        pl.BlockSpec((_BB, L), lambda i: (i, 0)),
        pl.BlockSpec((LANG_PAD, D), lambda i: (0, 0)),
    ]
    if prev is None:
        return pl.pallas_call(_finish_body0, in_specs=in_specs, **common)(
            flat_half, li_half, lproj
        )
    in_specs.append(pl.BlockSpec(memory_space=pl.ANY))
    return pl.pallas_call(
        _finish_body1,
        in_specs=in_specs,
        input_output_aliases={3: 0},
        **common,
    )(flat_half, li_half, lproj, prev)


def kernel(char_indices, lang_indices, char_table, lang_table, W, b):
    B, L = char_indices.shape
    n_total = B * L
    lang_padded = jnp.pad(lang_table, ((0, LANG_PAD - lang_table.shape[0]), (0, 0)))
    cproj, lproj = _project(char_table, lang_padded, W, b.reshape(1, D))
    ci = char_indices.reshape(-1).astype(jnp.int32)
    li = lang_indices.astype(jnp.int32)
    half_n = n_total // 2
    half_b = B // 2
    sc = _make_sc_gather(half_n)
    part0 = sc(cproj, ci[:half_n])
    part1 = sc(cproj, ci[half_n:])
    out0 = _finish_half(part0, li[:half_b], lproj, B, L, 0)
    out1 = _finish_half(part1, li[half_b:], lproj, B, L, 1, prev=out0)
    return out1


# speedup vs baseline: 1.3450x; 1.0391x over previous
import functools

import jax
import jax.numpy as jnp
from jax import lax
from jax.experimental import pallas as pl
from jax.experimental.pallas import tpu as pltpu
from jax.experimental.pallas import tpu_sc as plsc

D = 128
LANG_PAD = 104


def _project_body(char_ref, lang_ref, w_ref, b_ref, cout_ref, lout_ref):
    w = w_ref[...]
    w1 = w[:, :D]
    w2 = w[:, D:]
    cn = (((1,), (1,)), ((), ()))
    cout_ref[...] = (
        lax.dot_general(char_ref[...], w1, cn, preferred_element_type=jnp.float32)
        + b_ref[...]
    )
    lout_ref[...] = lax.dot_general(
        lang_ref[...], w2, cn, preferred_element_type=jnp.float32
    )


def _project(char_table, lang_table_padded, W, b2d):
    n_chars = char_table.shape[0]
    return pl.pallas_call(
        _project_body,
        out_shape=[
            jax.ShapeDtypeStruct((n_chars, D), jnp.float32),
            jax.ShapeDtypeStruct((LANG_PAD, D), jnp.float32),
        ],
    )(char_table, lang_table_padded, W, b2d)


def _make_sc_gather(n_total):
    info = plsc.get_sparse_core_info()
    nw = info.num_cores * info.num_subcores
    per_w = n_total // nw
    ch = 128
    n_ch = per_w // ch
    nbuf = 4
    mesh = plsc.VectorSubcoreMesh(core_axis_name="c", subcore_axis_name="s")

    @functools.partial(
        pl.kernel,
        mesh=mesh,
        out_type=jax.ShapeDtypeStruct((n_total, D), jnp.float32),
        scratch_types=[
            pltpu.VMEM((per_w,), jnp.int32),
            pltpu.VMEM((ch, D), jnp.float32),
            pltpu.VMEM((ch, D), jnp.float32),
            pltpu.VMEM((ch, D), jnp.float32),
            pltpu.VMEM((ch, D), jnp.float32),
            pltpu.SemaphoreType.DMA,
            pltpu.SemaphoreType.DMA,
            pltpu.SemaphoreType.DMA,
            pltpu.SemaphoreType.DMA,
            pltpu.SemaphoreType.DMA,
            pltpu.SemaphoreType.DMA,
            pltpu.SemaphoreType.DMA,
            pltpu.SemaphoreType.DMA,
        ],
    )
    def sc_gather(cproj_hbm, ci_hbm, out_hbm,
                  ci_v, ga0, ga1, ga2, ga3,
                  gs0, gs1, gs2, gs3, ss0, ss1, ss2, ss3):
        wid = lax.axis_index("s") * info.num_cores + lax.axis_index("c")
        base = wid * per_w
        pltpu.sync_copy(ci_hbm.at[pl.ds(base, per_w)], ci_v)
        gbufs = (ga0, ga1, ga2, ga3)
        gsems = (gs0, gs1, gs2, gs3)
        ssems = (ss0, ss1, ss2, ss3)

        def start_gather(g, b):
            pltpu.async_copy(
                cproj_hbm.at[ci_v.at[pl.ds(g * ch, ch)]], gbufs[b], gsems[b]
            )

        def wait_gather(b):
            pltpu.make_async_copy(
                cproj_hbm.at[ci_v.at[pl.ds(0, ch)]], gbufs[b], gsems[b]
            ).wait()

        def start_scatter(g, b):
            pltpu.async_copy(
                gbufs[b], out_hbm.at[pl.ds(base + g * ch, ch)], ssems[b]
            )

        def wait_scatter(b):
            pltpu.make_async_copy(
                gbufs[b], out_hbm.at[pl.ds(base, ch)], ssems[b]
            ).wait()

        for g in range(3):
            start_gather(g, g % nbuf)
        for t in range(n_ch):
            b = t % nbuf
            if t + 3 < n_ch:
                bn = (t + 3) % nbuf
                if t - 1 >= 0:
                    wait_scatter(bn)
                start_gather(t + 3, bn)
            wait_gather(b)
            start_scatter(t, b)
        for b in range(nbuf):
            wait_scatter(b)

    return sc_gather


_BB = 128


def _finish_math(part_ref, li_ref, lproj_ref, out_ref):
    L = li_ref.shape[1]
    li = li_ref[...]
    oh = (
        li[..., None] == lax.broadcasted_iota(jnp.int32, (_BB, L, LANG_PAD), 2)
    ).astype(jnp.float32)
    lproj = lproj_ref[...]
    lproj_hi = lproj.astype(jnp.bfloat16).astype(jnp.float32)
    lproj_lo = lproj - lproj_hi
    cn = (((2,), (0,)), ((), ()))
    lang_rows = lax.dot_general(
        oh, lproj_hi, cn, preferred_element_type=jnp.float32
    ) + lax.dot_general(oh, lproj_lo, cn, preferred_element_type=jnp.float32)
    out_ref[...] = part_ref[...].reshape(_BB, L, D) + lang_rows


def _finish_body0(part_ref, li_ref, lproj_ref, out_ref):
    _finish_math(part_ref, li_ref, lproj_ref, out_ref)


def _finish_body1(part_ref, li_ref, lproj_ref, prev_ref, out_ref):
    del prev_ref
    _finish_math(part_ref, li_ref, lproj_ref, out_ref)


def _finish_half(flat_half, li_half, lproj, B, L, half, prev=None):
    nblk = li_half.shape[0] // _BB
    off = half * nblk
    common = dict(
        grid=(nblk,),
        out_specs=pl.BlockSpec((_BB, L, D), lambda i: (i + off, 0, 0)),
        out_shape=jax.ShapeDtypeStruct((B, L, D), jnp.float32),
    )
    in_specs = [
        pl.BlockSpec((_BB * L, D), lambda i: (i, 0)),
        pl.BlockSpec((_BB, L), lambda i: (i, 0)),
        pl.BlockSpec((LANG_PAD, D), lambda i: (0, 0)),
    ]
    if prev is None:
        return pl.pallas_call(_finish_body0, in_specs=in_specs, **common)(
            flat_half, li_half, lproj
        )
    in_specs.append(pl.BlockSpec(memory_space=pl.ANY))
    return pl.pallas_call(
        _finish_body1,
        in_specs=in_specs,
        input_output_aliases={3: 0},
        **common,
    )(flat_half, li_half, lproj, prev)


def kernel(char_indices, lang_indices, char_table, lang_table, W, b):
    B, L = char_indices.shape
    n_total = B * L
    lang_padded = jnp.pad(lang_table, ((0, LANG_PAD - lang_table.shape[0]), (0, 0)))
    cproj, lproj = _project(char_table, lang_padded, W, b.reshape(1, D))
    ci = char_indices.reshape(-1).astype(jnp.int32)
    li = lang_indices.astype(jnp.int32)
    half_n = n_total // 2
    half_b = B // 2
    sc = _make_sc_gather(half_n)
    part0 = sc(cproj, ci[:half_n])
    part1 = sc(cproj, ci[half_n:])
    out0 = _finish_half(part0, li[:half_b], lproj, B, L, 0)
    out1 = _finish_half(part1, li[half_b:], lproj, B, L, 1, prev=out0)
    return out1


# speedup vs baseline: 1.3650x; 1.0149x over previous
import functools

import jax
import jax.numpy as jnp
from jax import lax
from jax.experimental import pallas as pl
from jax.experimental.pallas import tpu as pltpu
from jax.experimental.pallas import tpu_sc as plsc

D = 128
LANG_PAD = 104


def _project_body(char_ref, lang_ref, w_ref, b_ref, cout_ref, lout_ref):
    w = w_ref[...]
    w1 = w[:, :D]
    w2 = w[:, D:]
    cn = (((1,), (1,)), ((), ()))
    cout_ref[...] = (
        lax.dot_general(char_ref[...], w1, cn, preferred_element_type=jnp.float32)
        + b_ref[...]
    )
    lout_ref[...] = lax.dot_general(
        lang_ref[...], w2, cn, preferred_element_type=jnp.float32
    )


def _project(char_table, lang_table_padded, W, b2d):
    n_chars = char_table.shape[0]
    return pl.pallas_call(
        _project_body,
        out_shape=[
            jax.ShapeDtypeStruct((n_chars, D), jnp.float32),
            jax.ShapeDtypeStruct((LANG_PAD, D), jnp.float32),
        ],
    )(char_table, lang_table_padded, W, b2d)


def _make_sc_gather(n_total):
    info = plsc.get_sparse_core_info()
    nw = info.num_cores * info.num_subcores
    per_w = n_total // nw
    ch = 128
    n_ch = per_w // ch
    nbuf = 4
    mesh = plsc.VectorSubcoreMesh(core_axis_name="c", subcore_axis_name="s")

    @functools.partial(
        pl.kernel,
        mesh=mesh,
        out_type=jax.ShapeDtypeStruct((n_total, D), jnp.float32),
        scratch_types=[
            pltpu.VMEM((per_w,), jnp.int32),
            pltpu.VMEM((ch, D), jnp.float32),
            pltpu.VMEM((ch, D), jnp.float32),
            pltpu.VMEM((ch, D), jnp.float32),
            pltpu.VMEM((ch, D), jnp.float32),
            pltpu.SemaphoreType.DMA,
            pltpu.SemaphoreType.DMA,
            pltpu.SemaphoreType.DMA,
            pltpu.SemaphoreType.DMA,
            pltpu.SemaphoreType.DMA,
            pltpu.SemaphoreType.DMA,
            pltpu.SemaphoreType.DMA,
            pltpu.SemaphoreType.DMA,
        ],
    )
    def sc_gather(cproj_hbm, ci_hbm, out_hbm,
                  ci_v, ga0, ga1, ga2, ga3,
                  gs0, gs1, gs2, gs3, ss0, ss1, ss2, ss3):
        wid = lax.axis_index("s") * info.num_cores + lax.axis_index("c")
        base = wid * per_w
        pltpu.sync_copy(ci_hbm.at[pl.ds(base, per_w)], ci_v)
        gbufs = (ga0, ga1, ga2, ga3)
        gsems = (gs0, gs1, gs2, gs3)
        ssems = (ss0, ss1, ss2, ss3)

        def start_gather(g, b):
            pltpu.async_copy(
                cproj_hbm.at[ci_v.at[pl.ds(g * ch, ch)]], gbufs[b], gsems[b]
            )

        def wait_gather(b):
            pltpu.make_async_copy(
                cproj_hbm.at[ci_v.at[pl.ds(0, ch)]], gbufs[b], gsems[b]
            ).wait()

        def start_scatter(g, b):
            pltpu.async_copy(
                gbufs[b], out_hbm.at[pl.ds(base + g * ch, ch)], ssems[b]
            )

        def wait_scatter(b):
            pltpu.make_async_copy(
                gbufs[b], out_hbm.at[pl.ds(base, ch)], ssems[b]
            ).wait()

        for g in range(3):
            start_gather(g, g % nbuf)
        for t in range(n_ch):
            b = t % nbuf
            if t + 3 < n_ch:
                bn = (t + 3) % nbuf
                if t - 1 >= 0:
                    wait_scatter(bn)
                start_gather(t + 3, bn)
            wait_gather(b)
            start_scatter(t, b)
        for b in range(nbuf):
            wait_scatter(b)

    return sc_gather


_BB = 256


def _finish_math(part_ref, li_ref, lproj_ref, out_ref):
    L = li_ref.shape[1]
    li = li_ref[...]
    oh = (
        li[..., None] == lax.broadcasted_iota(jnp.int32, (_BB, L, LANG_PAD), 2)
    ).astype(jnp.float32)
    lproj = lproj_ref[...]
    lproj_hi = lproj.astype(jnp.bfloat16).astype(jnp.float32)
    lproj_lo = lproj - lproj_hi
    cn = (((2,), (0,)), ((), ()))
    lang_rows = lax.dot_general(
        oh, lproj_hi, cn, preferred_element_type=jnp.float32
    ) + lax.dot_general(oh, lproj_lo, cn, preferred_element_type=jnp.float32)
    out_ref[...] = part_ref[...].reshape(_BB, L, D) + lang_rows


def _finish_body0(part_ref, li_ref, lproj_ref, out_ref):
    _finish_math(part_ref, li_ref, lproj_ref, out_ref)


def _finish_body1(part_ref, li_ref, lproj_ref, prev_ref, out_ref):
    del prev_ref
    _finish_math(part_ref, li_ref, lproj_ref, out_ref)


def _finish_half(flat_half, li_half, lproj, B, L, half, prev=None):
    nblk = li_half.shape[0] // _BB
    off = half * nblk
    common = dict(
        grid=(nblk,),
        out_specs=pl.BlockSpec((_BB, L, D), lambda i: (i + off, 0, 0)),
        out_shape=jax.ShapeDtypeStruct((B, L, D), jnp.float32),
    )
    in_specs = [
        pl.BlockSpec((_BB * L, D), lambda i: (i, 0)),
        pl.BlockSpec((_BB, L), lambda i: (i, 0)),
        pl.BlockSpec((LANG_PAD, D), lambda i: (0, 0)),
    ]
    if prev is None:
        return pl.pallas_call(_finish_body0, in_specs=in_specs, **common)(
            flat_half, li_half, lproj
        )
    in_specs.append(pl.BlockSpec(memory_space=pl.ANY))
    return pl.pallas_call(
        _finish_body1,
        in_specs=in_specs,
        input_output_aliases={3: 0},
        **common,
    )(flat_half, li_half, lproj, prev)


def kernel(char_indices, lang_indices, char_table, lang_table, W, b):
    B, L = char_indices.shape
    n_total = B * L
    lang_padded = jnp.pad(lang_table, ((0, LANG_PAD - lang_table.shape[0]), (0, 0)))
    cproj, lproj = _project(char_table, lang_padded, W, b.reshape(1, D))
    ci = char_indices.reshape(-1).astype(jnp.int32)
    li = lang_indices.astype(jnp.int32)
    half_n = n_total // 2
    half_b = B // 2
    sc = _make_sc_gather(half_n)
    part0 = sc(cproj, ci[:half_n])
    part1 = sc(cproj, ci[half_n:])
    out0 = _finish_half(part0, li[:half_b], lproj, B, L, 0)
    out1 = _finish_half(part1, li[half_b:], lproj, B, L, 1, prev=out0)
    return out1
